# initial kernel scaffold (unmeasured)
import jax
import jax.numpy as jnp
from jax import lax
from jax.experimental import pallas as pl
from jax.experimental.pallas import tpu as pltpu

W = 16
M = 4096
N = 8192
MC = M // W

MESH = pl.DeviceIdType.MESH


def kernel(x, w_mat):
    k_shard = x.shape[1]

    def body(x_ref, w_ref, out_ref,
             xbf_ref, wbf_ref, send_buf, recv_buf, y_ref, amax_ref,
             ring_send_sems, ring_recv_sems,
             amax_send_sems, amax_recv_sems,
             credit_sem):
        me = lax.axis_index("i")
        left = (me + W - 1) % W
        right = (me + 1) % W

        barrier = pltpu.get_barrier_semaphore()
        pl.semaphore_signal(barrier, inc=1, device_id=(left,),
                            device_id_type=MESH)
        pl.semaphore_signal(barrier, inc=1, device_id=(right,),
                            device_id_type=MESH)
        pl.semaphore_wait(barrier, 2)

        xbf_ref[:, :] = x_ref[:, :].astype(jnp.bfloat16)
        wbf_ref[:, :] = w_ref[:, :].astype(jnp.bfloat16)

        def chunk_gemm(c):
            xc = xbf_ref[pl.ds(c * MC, MC), :]
            return lax.dot_general(
                xc, wbf_ref[:, :],
                (((1,), (0,)), ((), ())),
                preferred_element_type=jnp.float32,
            )

        for h in range(W - 1):
            slot = h % 2
            c = (me + (W - 1 - h)) % W
            g = chunk_gemm(c)
            if h == 0:
                acc = g
            else:
                acc = recv_buf[(h - 1) % 2].astype(jnp.float32) + g
                if h <= 13:
                    pl.semaphore_signal(credit_sem, inc=1,
                                        device_id=(left,),
                                        device_id_type=MESH)
            send_buf[slot, :, :] = acc.astype(jnp.bfloat16)
            if h >= 2:
                pl.semaphore_wait(credit_sem, 1)
            rdma = pltpu.make_async_remote_copy(
                src_ref=send_buf.at[slot],
                dst_ref=recv_buf.at[slot],
                send_sem=ring_send_sems.at[slot],
                recv_sem=ring_recv_sems.at[slot],
                device_id=(right,),
                device_id_type=MESH,
            )
            rdma.start()
            rdma.wait()

        y = recv_buf[(W - 2) % 2].astype(jnp.float32) + chunk_gemm(me)
        y_ref[:, :] = y

        amax = jnp.max(jnp.abs(y))
        amax_ref[pl.ds(me, 1), :] = jnp.full((1, 128), amax, jnp.float32)
        for q in range(W):
            @pl.when(q != me)
            def _():
                pltpu.make_async_remote_copy(
                    src_ref=amax_ref.at[pl.ds(me, 1)],
                    dst_ref=amax_ref.at[pl.ds(me, 1)],
                    send_sem=amax_send_sems.at[q],
                    recv_sem=amax_recv_sems.at[me],
                    device_id=(q,),
                    device_id_type=MESH,
                ).start()
        for q in range(W):
            @pl.when(q != me)
            def _():
                d = pltpu.make_async_remote_copy(
                    src_ref=amax_ref.at[pl.ds(q, 1)],
                    dst_ref=amax_ref.at[pl.ds(q, 1)],
                    send_sem=amax_send_sems.at[q],
                    recv_sem=amax_recv_sems.at[q],
                    device_id=(q,),
                    device_id_type=MESH,
                )
                d.wait_send()
                d.wait_recv()

        amax_all = jnp.max(amax_ref[:, :])
        scale = amax_all * (1.0 / 448.0)
        qv = (y_ref[:, :] * (1.0 / scale)).astype(jnp.float8_e4m3fn)
        out_ref[:, :] = qv.astype(jnp.float32) * scale

    return pl.pallas_call(
        body,
        out_shape=jax.ShapeDtypeStruct((MC, N), jnp.float32),
        in_specs=[
            pl.BlockSpec(memory_space=pltpu.VMEM),
            pl.BlockSpec(memory_space=pltpu.VMEM),
        ],
        out_specs=pl.BlockSpec(memory_space=pltpu.VMEM),
        scratch_shapes=[
            pltpu.VMEM((M, k_shard), jnp.bfloat16),
            pltpu.VMEM((k_shard, N), jnp.bfloat16),
            pltpu.VMEM((2, MC, N), jnp.bfloat16),
            pltpu.VMEM((2, MC, N), jnp.bfloat16),
            pltpu.VMEM((MC, N), jnp.float32),
            pltpu.VMEM((W, 128), jnp.float32),
            pltpu.SemaphoreType.DMA((2,)),
            pltpu.SemaphoreType.DMA((2,)),
            pltpu.SemaphoreType.DMA((W,)),
            pltpu.SemaphoreType.DMA((W,)),
            pltpu.SemaphoreType.REGULAR,
        ],
        compiler_params=pltpu.CompilerParams(
            collective_id=0,
            vmem_limit_bytes=100 * 1024 * 1024,
        ),
    )(x, w_mat)


# baseline (device time: 447745 ns/iter reference)
import jax
import jax.numpy as jnp
from jax import lax
from jax.experimental import pallas as pl
from jax.experimental.pallas import tpu as pltpu

W = 16
M = 4096
N = 8192
NH = N // 2
MC = M // W
S = 3

MESH = pl.DeviceIdType.MESH


def kernel(x, w_mat):
    k_shard = x.shape[1]

    def body(x_ref, w_ref, out_ref,
             xbf_ref, xrow_ref, wbfR_ref, wbfL_ref,
             bufR, bufL, y_ref, amax_ref,
             x_send_sems, x_recv_sems,
             initR_sem, initL_sem, fwdR_sem, fwdL_sem,
             recvR_sems, recvL_sems,
             amax_send_sems, amax_recv_sems,
             creditR_sem, creditL_sem):
        me = lax.axis_index("i")
        left = (me + W - 1) % W
        right = (me + 1) % W

        barrier = pltpu.get_barrier_semaphore()
        for q in range(W):
            @pl.when(q != me)
            def _():
                pl.semaphore_signal(barrier, inc=1, device_id=(q,),
                                    device_id_type=MESH)
        pl.semaphore_wait(barrier, W - 1)

        xbf_ref[:, :] = x_ref[:, :].astype(jnp.bfloat16)
        for q in range(W):
            @pl.when(q != me)
            def _():
                pltpu.make_async_remote_copy(
                    src_ref=xbf_ref.at[pl.ds(q * MC, MC)],
                    dst_ref=xrow_ref.at[:, pl.ds(me * k_shard, k_shard)],
                    send_sem=x_send_sems.at[q],
                    recv_sem=x_recv_sems.at[me],
                    device_id=(q,),
                    device_id_type=MESH,
                ).start()

        wbfR_ref[:, :] = w_ref[:, :NH].astype(jnp.bfloat16)
        wbfL_ref[:, :] = w_ref[:, NH:].astype(jnp.bfloat16)
        initR = pltpu.make_async_remote_copy(
            src_ref=wbfR_ref, dst_ref=bufR.at[0],
            send_sem=initR_sem, recv_sem=recvR_sems.at[0],
            device_id=(right,), device_id_type=MESH,
        )
        initR.start()
        initL = pltpu.make_async_remote_copy(
            src_ref=wbfL_ref, dst_ref=bufL.at[0],
            send_sem=initL_sem, recv_sem=recvL_sems.at[0],
            device_id=(left,), device_id_type=MESH,
        )
        initL.start()

        xown = xbf_ref[pl.ds(me * MC, MC), :]
        y_ref[:, :NH] = lax.dot_general(
            xown, wbfR_ref[:, :], (((1,), (0,)), ((), ())),
            preferred_element_type=jnp.float32)
        y_ref[:, NH:] = lax.dot_general(
            xown, wbfL_ref[:, :], (((1,), (0,)), ((), ())),
            preferred_element_type=jnp.float32)
        xrow_ref[:, pl.ds(me * k_shard, k_shard)] = xown

        for q in range(W):
            @pl.when(q != me)
            def _():
                pltpu.make_async_remote_copy(
                    src_ref=xbf_ref.at[pl.ds(q * MC, MC)],
                    dst_ref=xrow_ref.at[:, pl.ds(q * k_shard, k_shard)],
                    send_sem=x_send_sems.at[q],
                    recv_sem=x_recv_sems.at[q],
                    device_id=(q,),
                    device_id_type=MESH,
                ).wait_recv()

        def accum(origin, buf_ref, slot, col0):
            xo = xrow_ref[:, pl.ds(origin * k_shard, k_shard)]
            g = lax.dot_general(
                xo, buf_ref[slot], (((1,), (0,)), ((), ())),
                preferred_element_type=jnp.float32)
            y_ref[:, col0:col0 + NH] = y_ref[:, col0:col0 + NH] + g

        for h in range(W - 1):
            slot = h % S
            nslot = (h + 1) % S
            rR = pltpu.make_async_remote_copy(
                src_ref=bufR.at[slot], dst_ref=bufR.at[slot],
                send_sem=fwdR_sem, recv_sem=recvR_sems.at[slot],
                device_id=(right,), device_id_type=MESH,
            )
            rR.wait_recv()
            if h < W - 2:
                if h + 1 >= S:
                    pl.semaphore_wait(creditR_sem, 1)
                fR = pltpu.make_async_remote_copy(
                    src_ref=bufR.at[slot], dst_ref=bufR.at[nslot],
                    send_sem=fwdR_sem, recv_sem=recvR_sems.at[nslot],
                    device_id=(right,), device_id_type=MESH,
                )
                fR.start()
            rL = pltpu.make_async_remote_copy(
                src_ref=bufL.at[slot], dst_ref=bufL.at[slot],
                send_sem=fwdL_sem, recv_sem=recvL_sems.at[slot],
                device_id=(left,), device_id_type=MESH,
            )
            rL.wait_recv()
            if h < W - 2:
                if h + 1 >= S:
                    pl.semaphore_wait(creditL_sem, 1)
                fL = pltpu.make_async_remote_copy(
                    src_ref=bufL.at[slot], dst_ref=bufL.at[nslot],
                    send_sem=fwdL_sem, recv_sem=recvL_sems.at[nslot],
                    device_id=(left,), device_id_type=MESH,
                )
                fL.start()

            oR = (me + (W - 1 - h)) % W
            oL = (me + h + 1) % W
            accum(oR, bufR, slot, 0)
            accum(oL, bufL, slot, NH)

            if h < W - 2:
                fR.wait_send()
                fL.wait_send()
            if h + S <= W - 2:
                pl.semaphore_signal(creditR_sem, inc=1, device_id=(left,),
                                    device_id_type=MESH)
                pl.semaphore_signal(creditL_sem, inc=1, device_id=(right,),
                                    device_id_type=MESH)
            if h == 0:
                initR.wait_send()
                initL.wait_send()

        for q in range(W):
            @pl.when(q != me)
            def _():
                pltpu.make_async_remote_copy(
                    src_ref=xbf_ref.at[pl.ds(q * MC, MC)],
                    dst_ref=xrow_ref.at[:, pl.ds(q * k_shard, k_shard)],
                    send_sem=x_send_sems.at[q],
                    recv_sem=x_recv_sems.at[q],
                    device_id=(q,),
                    device_id_type=MESH,
                ).wait_send()

        amax = jnp.max(jnp.abs(y_ref[:, :]))
        amax_ref[pl.ds(me, 1), :] = jnp.full((1, 128), amax, jnp.float32)
        for q in range(W):
            @pl.when(q != me)
            def _():
                pltpu.make_async_remote_copy(
                    src_ref=amax_ref.at[pl.ds(me, 1)],
                    dst_ref=amax_ref.at[pl.ds(me, 1)],
                    send_sem=amax_send_sems.at[q],
                    recv_sem=amax_recv_sems.at[me],
                    device_id=(q,),
                    device_id_type=MESH,
                ).start()
        for q in range(W):
            @pl.when(q != me)
            def _():
                d = pltpu.make_async_remote_copy(
                    src_ref=amax_ref.at[pl.ds(q, 1)],
                    dst_ref=amax_ref.at[pl.ds(q, 1)],
                    send_sem=amax_send_sems.at[q],
                    recv_sem=amax_recv_sems.at[q],
                    device_id=(q,),
                    device_id_type=MESH,
                )
                d.wait_send()
                d.wait_recv()

        amax_all = jnp.max(amax_ref[:, :])
        scale = amax_all * (1.0 / 448.0)
        qv = (y_ref[:, :] * (1.0 / scale)).astype(jnp.float8_e4m3fn)
        out_ref[:, :] = qv.astype(jnp.float32) * scale

    return pl.pallas_call(
        body,
        out_shape=jax.ShapeDtypeStruct((MC, N), jnp.float32),
        in_specs=[
            pl.BlockSpec(memory_space=pltpu.VMEM),
            pl.BlockSpec(memory_space=pltpu.VMEM),
        ],
        out_specs=pl.BlockSpec(memory_space=pltpu.VMEM),
        scratch_shapes=[
            pltpu.VMEM((M, k_shard), jnp.bfloat16),
            pltpu.VMEM((MC, M), jnp.bfloat16),
            pltpu.VMEM((k_shard, NH), jnp.bfloat16),
            pltpu.VMEM((k_shard, NH), jnp.bfloat16),
            pltpu.VMEM((S, k_shard, NH), jnp.bfloat16),
            pltpu.VMEM((S, k_shard, NH), jnp.bfloat16),
            pltpu.VMEM((MC, N), jnp.float32),
            pltpu.VMEM((W, 128), jnp.float32),
            pltpu.SemaphoreType.DMA((W,)),
            pltpu.SemaphoreType.DMA((W,)),
            pltpu.SemaphoreType.DMA,
            pltpu.SemaphoreType.DMA,
            pltpu.SemaphoreType.DMA,
            pltpu.SemaphoreType.DMA,
            pltpu.SemaphoreType.DMA((S,)),
            pltpu.SemaphoreType.DMA((S,)),
            pltpu.SemaphoreType.DMA((W,)),
            pltpu.SemaphoreType.DMA((W,)),
            pltpu.SemaphoreType.REGULAR,
            pltpu.SemaphoreType.REGULAR,
        ],
        compiler_params=pltpu.CompilerParams(
            collective_id=0,
            vmem_limit_bytes=100 * 1024 * 1024,
        ),
    )(x, w_mat)


# device time: 432580 ns/iter; 1.0351x vs baseline; 1.0351x over previous
import jax
import jax.numpy as jnp
from jax import lax
from jax.experimental import pallas as pl
from jax.experimental.pallas import tpu as pltpu

W = 16
M = 4096
N = 8192
NQ = N // 4
MC = M // W
S = 3

MESH = pl.DeviceIdType.MESH


def kernel(x, w_mat):
    k_shard = x.shape[1]

    def body(x_ref, w_ref, out_ref,
             xbf_ref, xrow_ref, wbf_ref,
             buf0, buf1, buf2, buf3, y_ref, amax_ref,
             x_send_sems, x_recv_sems,
             init_sems, fwd_sems,
             recv0, recv1, recv2, recv3,
             amax_send_sems, amax_recv_sems,
             credit0, credit1, credit2, credit3):
        me = lax.axis_index("i")
        left = (me + W - 1) % W
        right = (me + 1) % W

        bufs = [buf0, buf1, buf2, buf3]
        recvs = [recv0, recv1, recv2, recv3]
        credits = [credit0, credit1, credit2, credit3]
        peer_out = [right, right, left, left]
        peer_cred = [left, left, right, right]

        barrier = pltpu.get_barrier_semaphore()
        for q in range(W):
            @pl.when(q != me)
            def _():
                pl.semaphore_signal(barrier, inc=1, device_id=(q,),
                                    device_id_type=MESH)
        pl.semaphore_wait(barrier, W - 1)

        xbf_ref[:, :] = x_ref[:, :].astype(jnp.bfloat16)
        for q in range(W):
            @pl.when(q != me)
            def _():
                pltpu.make_async_remote_copy(
                    src_ref=xbf_ref.at[pl.ds(q * MC, MC)],
                    dst_ref=xrow_ref.at[:, pl.ds(me * k_shard, k_shard)],
                    send_sem=x_send_sems.at[q],
                    recv_sem=x_recv_sems.at[me],
                    device_id=(q,),
                    device_id_type=MESH,
                ).start()

        wbf_ref[:, :] = w_ref[:, :].astype(jnp.bfloat16)
        inits = []
        for k in range(4):
            ik = pltpu.make_async_remote_copy(
                src_ref=wbf_ref.at[:, k * NQ:(k + 1) * NQ],
                dst_ref=bufs[k].at[0],
                send_sem=init_sems.at[k],
                recv_sem=recvs[k].at[0],
                device_id=(peer_out[k],),
                device_id_type=MESH,
            )
            ik.start()
            inits.append(ik)

        xown = xbf_ref[pl.ds(me * MC, MC), :]
        y_ref[:, :] = lax.dot_general(
            xown, wbf_ref[:, :], (((1,), (0,)), ((), ())),
            preferred_element_type=jnp.float32)
        xrow_ref[:, pl.ds(me * k_shard, k_shard)] = xown

        for q in range(W):
            @pl.when(q != me)
            def _():
                pltpu.make_async_remote_copy(
                    src_ref=xbf_ref.at[pl.ds(q * MC, MC)],
                    dst_ref=xrow_ref.at[:, pl.ds(q * k_shard, k_shard)],
                    send_sem=x_send_sems.at[q],
                    recv_sem=x_recv_sems.at[q],
                    device_id=(q,),
                    device_id_type=MESH,
                ).wait_recv()

        def accum(origin, buf_ref, slot, col0):
            xo = xrow_ref[:, pl.ds(origin * k_shard, k_shard)]
            g = lax.dot_general(
                xo, buf_ref[slot], (((1,), (0,)), ((), ())),
                preferred_element_type=jnp.float32)
            y_ref[:, col0:col0 + NQ] = y_ref[:, col0:col0 + NQ] + g

        for h in range(W - 1):
            slot = h % S
            nslot = (h + 1) % S
            fwds = []
            for k in range(4):
                pltpu.make_async_remote_copy(
                    src_ref=bufs[k].at[slot], dst_ref=bufs[k].at[slot],
                    send_sem=fwd_sems.at[k], recv_sem=recvs[k].at[slot],
                    device_id=(peer_out[k],), device_id_type=MESH,
                ).wait_recv()
                if h < W - 2:
                    if h + 1 >= S:
                        pl.semaphore_wait(credits[k], 1)
                    fk = pltpu.make_async_remote_copy(
                        src_ref=bufs[k].at[slot], dst_ref=bufs[k].at[nslot],
                        send_sem=fwd_sems.at[k], recv_sem=recvs[k].at[nslot],
                        device_id=(peer_out[k],), device_id_type=MESH,
                    )
                    fk.start()
                    fwds.append(fk)

            oR = (me + (W - 1 - h)) % W
            oL = (me + h + 1) % W
            accum(oR, bufs[0], slot, 0 * NQ)
            accum(oR, bufs[1], slot, 1 * NQ)
            accum(oL, bufs[2], slot, 2 * NQ)
            accum(oL, bufs[3], slot, 3 * NQ)

            for fk in fwds:
                fk.wait_send()
            if h + S <= W - 2:
                for k in range(4):
                    pl.semaphore_signal(credits[k], inc=1,
                                        device_id=(peer_cred[k],),
                                        device_id_type=MESH)
            if h == 0:
                for ik in inits:
                    ik.wait_send()

        for q in range(W):
            @pl.when(q != me)
            def _():
                pltpu.make_async_remote_copy(
                    src_ref=xbf_ref.at[pl.ds(q * MC, MC)],
                    dst_ref=xrow_ref.at[:, pl.ds(q * k_shard, k_shard)],
                    send_sem=x_send_sems.at[q],
                    recv_sem=x_recv_sems.at[q],
                    device_id=(q,),
                    device_id_type=MESH,
                ).wait_send()

        amax = jnp.max(jnp.abs(y_ref[:, :]))
        amax_ref[pl.ds(me, 1), :] = jnp.full((1, 128), amax, jnp.float32)
        for q in range(W):
            @pl.when(q != me)
            def _():
                pltpu.make_async_remote_copy(
                    src_ref=amax_ref.at[pl.ds(me, 1)],
                    dst_ref=amax_ref.at[pl.ds(me, 1)],
                    send_sem=amax_send_sems.at[q],
                    recv_sem=amax_recv_sems.at[me],
                    device_id=(q,),
                    device_id_type=MESH,
                ).start()
        for q in range(W):
            @pl.when(q != me)
            def _():
                d = pltpu.make_async_remote_copy(
                    src_ref=amax_ref.at[pl.ds(q, 1)],
                    dst_ref=amax_ref.at[pl.ds(q, 1)],
                    send_sem=amax_send_sems.at[q],
                    recv_sem=amax_recv_sems.at[q],
                    device_id=(q,),
                    device_id_type=MESH,
                )
                d.wait_send()
                d.wait_recv()

        amax_all = jnp.max(amax_ref[:, :])
        scale = amax_all * (1.0 / 448.0)
        qv = (y_ref[:, :] * (1.0 / scale)).astype(jnp.float8_e4m3fn)
        out_ref[:, :] = qv.astype(jnp.float32) * scale

    return pl.pallas_call(
        body,
        out_shape=jax.ShapeDtypeStruct((MC, N), jnp.float32),
        in_specs=[
            pl.BlockSpec(memory_space=pltpu.VMEM),
            pl.BlockSpec(memory_space=pltpu.VMEM),
        ],
        out_specs=pl.BlockSpec(memory_space=pltpu.VMEM),
        scratch_shapes=[
            pltpu.VMEM((M, k_shard), jnp.bfloat16),
            pltpu.VMEM((MC, M), jnp.bfloat16),
            pltpu.VMEM((k_shard, N), jnp.bfloat16),
            pltpu.VMEM((S, k_shard, NQ), jnp.bfloat16),
            pltpu.VMEM((S, k_shard, NQ), jnp.bfloat16),
            pltpu.VMEM((S, k_shard, NQ), jnp.bfloat16),
            pltpu.VMEM((S, k_shard, NQ), jnp.bfloat16),
            pltpu.VMEM((MC, N), jnp.float32),
            pltpu.VMEM((W, 128), jnp.float32),
            pltpu.SemaphoreType.DMA((W,)),
            pltpu.SemaphoreType.DMA((W,)),
            pltpu.SemaphoreType.DMA((4,)),
            pltpu.SemaphoreType.DMA((4,)),
            pltpu.SemaphoreType.DMA((S,)),
            pltpu.SemaphoreType.DMA((S,)),
            pltpu.SemaphoreType.DMA((S,)),
            pltpu.SemaphoreType.DMA((S,)),
            pltpu.SemaphoreType.DMA((W,)),
            pltpu.SemaphoreType.DMA((W,)),
            pltpu.SemaphoreType.REGULAR,
            pltpu.SemaphoreType.REGULAR,
            pltpu.SemaphoreType.REGULAR,
            pltpu.SemaphoreType.REGULAR,
        ],
        compiler_params=pltpu.CompilerParams(
            collective_id=0,
            vmem_limit_bytes=100 * 1024 * 1024,
        ),
    )(x, w_mat)


# device time: 415166 ns/iter; 1.0785x vs baseline; 1.0419x over previous
import jax
import jax.numpy as jnp
from jax import lax
from jax.experimental import pallas as pl
from jax.experimental.pallas import tpu as pltpu

W = 16
M = 4096
N = 8192
NQ = N // 4
MC = M // W
S = 3

MESH = pl.DeviceIdType.MESH


def kernel(x, w_mat):
    k_shard = x.shape[1]

    def body(x_ref, w_ref, out_ref,
             xbf_ref, xrow_ref, wbf_ref,
             buf0, buf1, buf2, buf3, y_ref, amax_ref,
             x_send_sems, x_recv_sems,
             init_sems, fwd_sems,
             recv0, recv1, recv2, recv3,
             amax_send_sems, amax_recv_sems,
             credit0, credit1, credit2, credit3):
        me = lax.axis_index("i")
        left = (me + W - 1) % W
        right = (me + 1) % W

        bufs = [buf0, buf1, buf2, buf3]
        recvs = [recv0, recv1, recv2, recv3]
        credits = [credit0, credit1, credit2, credit3]
        peer_out = [right, right, left, left]
        peer_cred = [left, left, right, right]

        barrier = pltpu.get_barrier_semaphore()
        for q in range(W):
            @pl.when(q != me)
            def _():
                pl.semaphore_signal(barrier, inc=1, device_id=(q,),
                                    device_id_type=MESH)
        pl.semaphore_wait(barrier, W - 1)

        wbf_ref[:, :] = w_ref[:, :].astype(jnp.bfloat16)
        inits = []
        for k in range(4):
            ik = pltpu.make_async_remote_copy(
                src_ref=wbf_ref.at[:, k * NQ:(k + 1) * NQ],
                dst_ref=bufs[k].at[0],
                send_sem=init_sems.at[k],
                recv_sem=recvs[k].at[0],
                device_id=(peer_out[k],),
                device_id_type=MESH,
            )
            ik.start()
            inits.append(ik)

        xbf_ref[:, :] = x_ref[:, :].astype(jnp.bfloat16)
        for q in range(W):
            @pl.when(q != me)
            def _():
                pltpu.make_async_remote_copy(
                    src_ref=xbf_ref.at[pl.ds(q * MC, MC)],
                    dst_ref=xrow_ref.at[:, pl.ds(me * k_shard, k_shard)],
                    send_sem=x_send_sems.at[q],
                    recv_sem=x_recv_sems.at[me],
                    device_id=(q,),
                    device_id_type=MESH,
                ).start()

        xown = xbf_ref[pl.ds(me * MC, MC), :]
        y_ref[:, :] = lax.dot_general(
            xown, wbf_ref[:, :], (((1,), (0,)), ((), ())),
            preferred_element_type=jnp.float32)
        xrow_ref[:, pl.ds(me * k_shard, k_shard)] = xown

        for q in range(W):
            @pl.when(q != me)
            def _():
                pltpu.make_async_remote_copy(
                    src_ref=xbf_ref.at[pl.ds(q * MC, MC)],
                    dst_ref=xrow_ref.at[:, pl.ds(q * k_shard, k_shard)],
                    send_sem=x_send_sems.at[q],
                    recv_sem=x_recv_sems.at[q],
                    device_id=(q,),
                    device_id_type=MESH,
                ).wait_recv()

        def accum(origin, buf_ref, slot, col0):
            xo = xrow_ref[:, pl.ds(origin * k_shard, k_shard)]
            g = lax.dot_general(
                xo, buf_ref[slot], (((1,), (0,)), ((), ())),
                preferred_element_type=jnp.float32)
            y_ref[:, col0:col0 + NQ] = y_ref[:, col0:col0 + NQ] + g

        pending = [None, None, None, None]
        for h in range(W - 1):
            slot = h % S
            nslot = (h + 1) % S
            for k in (0, 2, 1, 3):
                pltpu.make_async_remote_copy(
                    src_ref=bufs[k].at[slot], dst_ref=bufs[k].at[slot],
                    send_sem=fwd_sems.at[k], recv_sem=recvs[k].at[slot],
                    device_id=(peer_out[k],), device_id_type=MESH,
                ).wait_recv()
                if pending[k] is not None:
                    fk_prev, h_prev = pending[k]
                    fk_prev.wait_send()
                    pending[k] = None
                    if h_prev + S <= W - 2:
                        pl.semaphore_signal(credits[k], inc=1,
                                            device_id=(peer_cred[k],),
                                            device_id_type=MESH)
                if h < W - 2:
                    if h + 1 >= S:
                        pl.semaphore_wait(credits[k], 1)
                    fk = pltpu.make_async_remote_copy(
                        src_ref=bufs[k].at[slot], dst_ref=bufs[k].at[nslot],
                        send_sem=fwd_sems.at[k], recv_sem=recvs[k].at[nslot],
                        device_id=(peer_out[k],), device_id_type=MESH,
                    )
                    fk.start()
                    pending[k] = (fk, h)

            oR = (me + (W - 1 - h)) % W
            oL = (me + h + 1) % W
            accum(oR, bufs[0], slot, 0 * NQ)
            accum(oR, bufs[1], slot, 1 * NQ)
            accum(oL, bufs[2], slot, 2 * NQ)
            accum(oL, bufs[3], slot, 3 * NQ)

        for ik in inits:
            ik.wait_send()

        for q in range(W):
            @pl.when(q != me)
            def _():
                pltpu.make_async_remote_copy(
                    src_ref=xbf_ref.at[pl.ds(q * MC, MC)],
                    dst_ref=xrow_ref.at[:, pl.ds(q * k_shard, k_shard)],
                    send_sem=x_send_sems.at[q],
                    recv_sem=x_recv_sems.at[q],
                    device_id=(q,),
                    device_id_type=MESH,
                ).wait_send()

        amax = jnp.max(jnp.abs(y_ref[:, :]))
        amax_ref[pl.ds(me, 1), :] = jnp.full((1, 128), amax, jnp.float32)
        for q in range(W):
            @pl.when(q != me)
            def _():
                pltpu.make_async_remote_copy(
                    src_ref=amax_ref.at[pl.ds(me, 1)],
                    dst_ref=amax_ref.at[pl.ds(me, 1)],
                    send_sem=amax_send_sems.at[q],
                    recv_sem=amax_recv_sems.at[me],
                    device_id=(q,),
                    device_id_type=MESH,
                ).start()
        for q in range(W):
            @pl.when(q != me)
            def _():
                d = pltpu.make_async_remote_copy(
                    src_ref=amax_ref.at[pl.ds(q, 1)],
                    dst_ref=amax_ref.at[pl.ds(q, 1)],
                    send_sem=amax_send_sems.at[q],
                    recv_sem=amax_recv_sems.at[q],
                    device_id=(q,),
                    device_id_type=MESH,
                )
                d.wait_send()
                d.wait_recv()

        amax_all = jnp.max(amax_ref[:, :])
        scale = amax_all * (1.0 / 448.0)
        qv = (y_ref[:, :] * (1.0 / scale)).astype(jnp.float8_e4m3fn)
        out_ref[:, :] = qv.astype(jnp.float32) * scale

    return pl.pallas_call(
        body,
        out_shape=jax.ShapeDtypeStruct((MC, N), jnp.float32),
        in_specs=[
            pl.BlockSpec(memory_space=pltpu.VMEM),
            pl.BlockSpec(memory_space=pltpu.VMEM),
        ],
        out_specs=pl.BlockSpec(memory_space=pltpu.VMEM),
        scratch_shapes=[
            pltpu.VMEM((M, k_shard), jnp.bfloat16),
            pltpu.VMEM((MC, M), jnp.bfloat16),
            pltpu.VMEM((k_shard, N), jnp.bfloat16),
            pltpu.VMEM((S, k_shard, NQ), jnp.bfloat16),
            pltpu.VMEM((S, k_shard, NQ), jnp.bfloat16),
            pltpu.VMEM((S, k_shard, NQ), jnp.bfloat16),
            pltpu.VMEM((S, k_shard, NQ), jnp.bfloat16),
            pltpu.VMEM((MC, N), jnp.float32),
            pltpu.VMEM((W, 128), jnp.float32),
            pltpu.SemaphoreType.DMA((W,)),
            pltpu.SemaphoreType.DMA((W,)),
            pltpu.SemaphoreType.DMA((4,)),
            pltpu.SemaphoreType.DMA((4,)),
            pltpu.SemaphoreType.DMA((S,)),
            pltpu.SemaphoreType.DMA((S,)),
            pltpu.SemaphoreType.DMA((S,)),
            pltpu.SemaphoreType.DMA((S,)),
            pltpu.SemaphoreType.DMA((W,)),
            pltpu.SemaphoreType.DMA((W,)),
            pltpu.SemaphoreType.REGULAR,
            pltpu.SemaphoreType.REGULAR,
            pltpu.SemaphoreType.REGULAR,
            pltpu.SemaphoreType.REGULAR,
        ],
        compiler_params=pltpu.CompilerParams(
            collective_id=0,
            vmem_limit_bytes=100 * 1024 * 1024,
        ),
    )(x, w_mat)
